# baseline (speedup 1.0000x reference)
import functools
import math

import jax
import jax.numpy as jnp
from jax import lax
from jax.experimental import pallas as pl
from jax.experimental.pallas import tpu as pltpu
from jax.experimental.pallas import tpu_sc as plsc

B, L, D = 1, 8192, 1024
H = 16
DH = D // H
INNER = 1024
NUM_PROJS = 7
BLOCK = 256
SAMPLE = 256
NB = L // BLOCK
NCH = 32
CH = L // NCH
SCALE = DH ** -0.5
ROW = 2 * DH

_BF = jnp.bfloat16
_F32 = jnp.float32

_NC = 2
_LN = 16
GCH = 128


def _dot_bf16(a, b):
    return jnp.dot(a.astype(_BF), b.astype(_BF), preferred_element_type=_F32)


def _dotg_bf16(a, b):
    return lax.dot_general(a.astype(_BF), b.astype(_BF),
                           (((1,), (1,)), ((), ())),
                           preferred_element_type=_F32)


def _bins_from(s):
    lane = lax.broadcasted_iota(jnp.int32, (1, 128), 1)
    enc = jnp.where(lane < NUM_PROJS,
                    jnp.left_shift(1, jnp.where(lane < NUM_PROJS, lane, 0)), 0)
    bins = jnp.sum(jnp.where(s > 0, enc, 0), axis=1)
    return jnp.bitwise_xor(bins, jnp.right_shift(bins, 1))


def _qkv_kernel(x_ref, w_ref, b_ref, lsh_ref, oq_ref, okv_ref,
                obq_ref, obk_ref):
    h = pl.program_id(1)
    x = x_ref[...]
    zpad = jnp.zeros((x.shape[0], DH), _F32)

    def head_dot(sec):
        w = w_ref[pl.ds(sec * INNER + h * DH, DH), :]
        b = b_ref[pl.ds(sec * H + h, 1), :]
        return _dotg_bf16(x, w) + b

    q64 = head_dot(0)
    k64 = head_dot(1)
    v64 = head_dot(2)
    oq_ref[0, :, :DH] = q64
    oq_ref[0, :, DH:] = zpad
    okv_ref[0, :, :DH] = k64
    okv_ref[0, :, DH:] = v64
    obq_ref[0, 0, :] = _bins_from(_dot_bf16(q64, lsh_ref[...]))
    obk_ref[0, 0, :] = _bins_from(_dot_bf16(k64, lsh_ref[...]))


def _qkv_tables(x2d, w_bf, b3, lsh64):
    BM = 2048
    ob = pl.BlockSpec((1, BM, ROW), lambda i, h: (h, i, 0))
    bins_b = pl.BlockSpec((1, 1, BM), lambda i, h: (h * (L // BM) + i, 0, 0))
    return pl.pallas_call(
        _qkv_kernel,
        grid=(L // BM, H),
        in_specs=[pl.BlockSpec((BM, D), lambda i, h: (i, 0)),
                  pl.BlockSpec((3 * INNER, D), lambda i, h: (0, 0)),
                  pl.BlockSpec((3 * H, DH), lambda i, h: (0, 0)),
                  pl.BlockSpec((DH, 128), lambda i, h: (0, 0))],
        out_specs=[ob, ob, bins_b, bins_b],
        out_shape=[jax.ShapeDtypeStruct((H, L, ROW), _F32),
                   jax.ShapeDtypeStruct((H, L, ROW), _F32),
                   jax.ShapeDtypeStruct((H * L // BM, 1, BM), jnp.int32),
                   jax.ShapeDtypeStruct((H * L // BM, 1, BM), jnp.int32)],
    )(x2d, w_bf, b3, lsh64)


_T = NCH * 128


def _sortpos_kernel(keys_ref, tri256_ref, e_ref, r_ref, v_ref, v2_ref,
                    triu32_ref, tri128_ref, vpat_ref, o_ref):
    keysf = keys_ref[0].astype(_F32)
    krep = jnp.dot(keysf.astype(_BF), e_ref[...],
                   preferred_element_type=_F32)
    oh = (krep == vpat_ref[...]).astype(_F32)
    ohb = oh.astype(_BF)
    rank_m = jnp.dot(tri256_ref[...], ohb, preferred_element_type=_F32)
    rank = jnp.dot((oh * rank_m).astype(_BF), r_ref[...],
                   preferred_element_type=_F32)
    hist_t = jnp.sum(oh, axis=0, keepdims=True)
    hist_vc = jnp.dot((v2_ref[...] * hist_t).astype(_BF), r_ref[...],
                      preferred_element_type=_F32)
    excl_vc = jnp.dot(hist_vc.astype(_BF), triu32_ref[...],
                      preferred_element_type=_F32)
    totalv = jnp.sum(hist_vc, axis=1, keepdims=True)
    key_off = jnp.dot(tri128_ref[...], totalv,
                      precision=lax.Precision.HIGHEST,
                      preferred_element_type=_F32)
    base_vc = key_off + excl_vc
    hi = jnp.floor(base_vc * (1.0 / 32.0))
    lo = base_vc - hi * 32.0
    bhi = jnp.dot(v_ref[...], hi.astype(_BF), preferred_element_type=_F32)
    blo = jnp.dot(v_ref[...], lo.astype(_BF), preferred_element_type=_F32)
    r_f = r_ref[...].astype(_F32)
    ph = jnp.dot(ohb, (bhi * r_f).astype(_BF), preferred_element_type=_F32)
    pll = jnp.dot(ohb, (blo * r_f).astype(_BF), preferred_element_type=_F32)
    pos = ph * 32.0 + pll + rank
    o_ref[0] = pos.astype(jnp.int32)


def _sort_positions(keysT):
    t_row = lax.broadcasted_iota(jnp.int32, (1, _T), 1)
    c_of_t = t_row >> 7
    v_of_t = t_row & 127
    e_m = ((t_row >> 7) == lax.broadcasted_iota(jnp.int32, (NCH, 1), 0)).astype(_BF)
    t_col = lax.broadcasted_iota(jnp.int32, (_T, 1), 0)
    r_m = ((t_col >> 7) == lax.broadcasted_iota(jnp.int32, (1, NCH), 1)).astype(_BF)
    v_m = ((t_col & 127) == lax.broadcasted_iota(jnp.int32, (1, 128), 1)).astype(_BF)
    v2_m = ((t_row & 127) == lax.broadcasted_iota(jnp.int32, (128, 1), 0)).astype(_F32)
    i256 = lax.broadcasted_iota(jnp.int32, (CH, CH), 0)
    j256 = lax.broadcasted_iota(jnp.int32, (CH, CH), 1)
    tri256 = (j256 < i256).astype(_BF)
    i32m = lax.broadcasted_iota(jnp.int32, (NCH, NCH), 0)
    j32m = lax.broadcasted_iota(jnp.int32, (NCH, NCH), 1)
    triu32 = (i32m < j32m).astype(_BF)
    i128 = lax.broadcasted_iota(jnp.int32, (128, 128), 0)
    j128 = lax.broadcasted_iota(jnp.int32, (128, 128), 1)
    tri128 = (j128 < i128).astype(_F32)
    vpat = v_of_t.astype(_F32)

    full = lambda shape: pl.BlockSpec(shape, lambda i: tuple(0 for _ in shape))
    return pl.pallas_call(
        _sortpos_kernel,
        grid=(2 * H,),
        in_specs=[pl.BlockSpec((1, CH, NCH), lambda i: (i, 0, 0)),
                  full((CH, CH)), full((NCH, _T)), full((_T, NCH)),
                  full((_T, 128)), full((128, _T)), full((NCH, NCH)),
                  full((128, 128)), full((1, _T))],
        out_specs=pl.BlockSpec((1, CH, NCH), lambda i: (i, 0, 0)),
        out_shape=jax.ShapeDtypeStruct((2 * H, CH, NCH), jnp.int32),
    )(keysT, tri256, e_m, r_m, v_m, v2_m, triu32, tri128, vpat)


def _sc_gather_sorted(table_q, table_kv, pos_flat, samp_flat):
    mesh = plsc.VectorSubcoreMesh(core_axis_name="c", subcore_axis_name="s")

    @functools.partial(
        pl.kernel, mesh=mesh,
        out_type=[jax.ShapeDtypeStruct((H, L, ROW), _F32),
                  jax.ShapeDtypeStruct((H, L, ROW), _F32),
                  jax.ShapeDtypeStruct((H, SAMPLE, ROW), _F32)],
        scratch_types=[pltpu.VMEM((L,), jnp.int32),
                       pltpu.VMEM((GCH,), jnp.int32),
                       pltpu.VMEM((GCH,), jnp.int32),
                       pltpu.VMEM((GCH, ROW), _F32),
                       pltpu.VMEM((GCH, ROW), _F32),
                       pltpu.VMEM((SAMPLE,), jnp.int32),
                       pltpu.SemaphoreType.DMA,
                       pltpu.SemaphoreType.DMA,
                       pltpu.SemaphoreType.DMA,
                       pltpu.SemaphoreType.DMA],
    )
    def k(tq_hbm, tkv_hbm, pos_hbm, samp_hbm, qs_hbm, kvs_hbm, sub_hbm,
          posbuf, idx_c0, idx_c1, rowbuf0, rowbuf1, sampbuf,
          semr0, semr1, semw0, semw1):
        wid = lax.axis_index("s") * _NC + lax.axis_index("c")
        h = lax.rem(wid, H)

        pltpu.sync_copy(pos_hbm.at[pl.ds(wid * L, L)], posbuf)

        def scatter_pair(table, out, j):
            c0, c1 = 2 * j, 2 * j + 1
            r0 = pltpu.async_copy(table.at[h].at[pl.ds(c0 * GCH, GCH)],
                                  rowbuf0, semr0)
            r1 = pltpu.async_copy(table.at[h].at[pl.ds(c1 * GCH, GCH)],
                                  rowbuf1, semr1)
            for r in range(GCH // _LN):
                idx_c0[pl.ds(r * _LN, _LN)] = \
                    posbuf[pl.ds(c0 * GCH + r * _LN, _LN)]
                idx_c1[pl.ds(r * _LN, _LN)] = \
                    posbuf[pl.ds(c1 * GCH + r * _LN, _LN)]
            r0.wait()
            w0 = pltpu.async_copy(rowbuf0, out.at[h].at[idx_c0], semw0)
            r1.wait()
            w1 = pltpu.async_copy(rowbuf1, out.at[h].at[idx_c1], semw1)
            w0.wait()
            w1.wait()

        @pl.when(wid < H)
        def _():
            lax.fori_loop(0, L // GCH // 2,
                          lambda j, _: (scatter_pair(tq_hbm, qs_hbm, j), 0)[1],
                          0)

        @pl.when(wid >= H)
        def _():
            lax.fori_loop(0, L // GCH // 2,
                          lambda j, _: (scatter_pair(tkv_hbm, kvs_hbm, j), 0)[1],
                          0)
            pltpu.sync_copy(samp_hbm.at[pl.ds(h * SAMPLE, SAMPLE)], sampbuf)
            for cc in range(SAMPLE // GCH):
                for r in range(GCH // _LN):
                    idx_c0[pl.ds(r * _LN, _LN)] = \
                        sampbuf[pl.ds(cc * GCH + r * _LN, _LN)]
                pltpu.async_copy(kvs_hbm.at[h].at[idx_c0], rowbuf0, semr0).wait()
                pltpu.sync_copy(rowbuf0, sub_hbm.at[h].at[pl.ds(cc * GCH, GCH)])

    return k(table_q, table_kv, pos_flat, samp_flat)


def _sc_unsort(attn, pos_flat):
    mesh = plsc.VectorSubcoreMesh(core_axis_name="c", subcore_axis_name="s")
    HALF = L // 2

    @functools.partial(
        pl.kernel, mesh=mesh,
        out_type=jax.ShapeDtypeStruct((H, L, ROW), _F32),
        scratch_types=[pltpu.VMEM((HALF,), jnp.int32),
                       pltpu.VMEM((GCH,), jnp.int32),
                       pltpu.VMEM((GCH,), jnp.int32),
                       pltpu.VMEM((GCH, ROW), _F32),
                       pltpu.VMEM((GCH, ROW), _F32),
                       pltpu.SemaphoreType.DMA,
                       pltpu.SemaphoreType.DMA,
                       pltpu.SemaphoreType.DMA,
                       pltpu.SemaphoreType.DMA],
    )
    def k(attn_hbm, pos_hbm, out_hbm, posbuf, idx_c0, idx_c1,
          rowbuf0, rowbuf1, semr0, semr1, semw0, semw1):
        wid = lax.axis_index("s") * _NC + lax.axis_index("c")
        h = lax.rem(wid, H)
        half = wid // H
        base = h * L + half * HALF
        pltpu.sync_copy(pos_hbm.at[pl.ds(base, HALF)], posbuf)

        def body(j, _):
            c0, c1 = 2 * j, 2 * j + 1
            for r in range(GCH // _LN):
                idx_c0[pl.ds(r * _LN, _LN)] = \
                    posbuf[pl.ds(c0 * GCH + r * _LN, _LN)]
                idx_c1[pl.ds(r * _LN, _LN)] = \
                    posbuf[pl.ds(c1 * GCH + r * _LN, _LN)]
            r0 = pltpu.async_copy(attn_hbm.at[h].at[idx_c0], rowbuf0, semr0)
            r1 = pltpu.async_copy(attn_hbm.at[h].at[idx_c1], rowbuf1, semr1)
            r0.wait()
            w0 = pltpu.async_copy(
                rowbuf0, out_hbm.at[h].at[pl.ds(half * HALF + c0 * GCH, GCH)],
                semw0)
            r1.wait()
            w1 = pltpu.async_copy(
                rowbuf1, out_hbm.at[h].at[pl.ds(half * HALF + c1 * GCH, GCH)],
                semw1)
            w0.wait()
            w1.wait()
            return 0
        lax.fori_loop(0, HALF // GCH // 2, body, 0)

    return k(attn, pos_flat)


_AB = 8


def _attn_kernel(q_ref, kv_ref, sub_ref, samp_ref, o_ref):
    sub = sub_ref[0]
    samp = samp_ref[0]
    sampb = samp // BLOCK
    for b in range(_AB):
        nb = pl.program_id(1) * _AB + b
        sl = pl.ds(b * BLOCK, BLOCK)
        qb = q_ref[0, sl, :DH]
        kv = kv_ref[0, sl, :]
        kb, vb = kv[:, :DH], kv[:, DH:]

        s1 = _dotg_bf16(qb, kb) * SCALE
        m1 = jnp.max(s1, axis=1, keepdims=True)
        e1 = jnp.exp(s1 - m1)
        l1 = jnp.log(jnp.sum(e1, axis=1, keepdims=True)) + m1
        a1 = jnp.exp(s1 - l1)
        o1 = _dot_bf16(a1, vb)

        bias = (sampb == nb).astype(_F32) * jnp.finfo(_F32).min
        s2 = _dotg_bf16(qb, sub[:, :DH]) * SCALE + bias
        m2 = jnp.max(s2, axis=1, keepdims=True)
        e2 = jnp.exp(s2 - m2)
        l2 = jnp.log(jnp.sum(e2, axis=1, keepdims=True)) + m2
        a2 = jnp.exp(s2 - l2)
        o2 = _dot_bf16(a2, sub[:, DH:])

        lse_res = l2 + math.log(L / SAMPLE)
        c = 1.0 / (1.0 + jnp.exp(lse_res - l1))
        o_ref[0, sl, :DH] = c * o1 + (1.0 - c) * o2
        o_ref[0, sl, DH:] = jnp.zeros((BLOCK, DH), _F32)


def _fused_attention(q_s, kv_s, kv_sub, samp):
    blk = pl.BlockSpec((1, _AB * BLOCK, ROW), lambda h, n: (h, n, 0))
    head = pl.BlockSpec((1, SAMPLE, ROW), lambda h, n: (h, 0, 0))
    return pl.pallas_call(
        _attn_kernel,
        grid=(H, NB // _AB),
        in_specs=[blk, blk, head,
                  pl.BlockSpec((1, 1, SAMPLE), lambda h, n: (h, 0, 0))],
        out_specs=blk,
        out_shape=jax.ShapeDtypeStruct((H, L, ROW), _F32),
    )(q_s, kv_s, kv_sub, samp)


def _proj_kernel(a_ref, w_ref, b_ref, o_ref):
    acc = _dot_bf16(a_ref[0], w_ref[0])
    for h in range(1, H):
        acc = acc + _dot_bf16(a_ref[h], w_ref[h])
    o_ref[...] = acc + b_ref[...]


def _proj_matmul(attn_u, w2, pb):
    BM = 1024
    return pl.pallas_call(
        _proj_kernel,
        grid=(L // BM,),
        in_specs=[pl.BlockSpec((H, BM, ROW), lambda i: (0, i, 0)),
                  pl.BlockSpec((H, ROW, D), lambda i: (0, 0, 0)),
                  pl.BlockSpec((1, D), lambda i: (0, 0))],
        out_specs=pl.BlockSpec((BM, D), lambda i: (i, 0)),
        out_shape=jax.ShapeDtypeStruct((L, D), _F32),
    )(attn_u, w2, pb)


def kernel(x, qkv_w, qkv_b, proj_w, proj_b, lsh_proj, sampled_set):
    x2d = x.reshape(L, D)

    w_bf = qkv_w.astype(_BF)
    b3 = qkv_b.reshape(3 * H, DH)
    lsh64 = jnp.concatenate(
        [lsh_proj, jnp.zeros((DH, 128 - NUM_PROJS), _F32)], axis=1)
    w2 = jnp.concatenate(
        [proj_w.T.reshape(H, DH, D), jnp.zeros((H, DH, D), _F32)],
        axis=1).astype(_BF)

    table_q, table_kv, bins_q, bins_k = _qkv_tables(x2d, w_bf, b3, lsh64)
    bins = jnp.concatenate([bins_q, bins_k], axis=0)
    keysT = bins.reshape(2 * H, NCH, CH).transpose(0, 2, 1)
    posT = _sort_positions(keysT)
    pos_flat = posT.transpose(0, 2, 1).reshape(2 * H * L)

    samp = sampled_set[0].astype(jnp.int32)
    samp_flat = samp.reshape(H * SAMPLE)

    q_s, kv_s, kv_sub = _sc_gather_sorted(table_q, table_kv,
                                          pos_flat, samp_flat)

    attn = _fused_attention(q_s, kv_s, kv_sub, samp.reshape(H, 1, SAMPLE))

    attn_u = _sc_unsort(attn, pos_flat)

    out = _proj_matmul(attn_u, w2, proj_b.reshape(1, D))
    return out.reshape(B, L, D)


# speedup vs baseline: 1.1411x; 1.1411x over previous
import functools
import math

import jax
import jax.numpy as jnp
from jax import lax
from jax.experimental import pallas as pl
from jax.experimental.pallas import tpu as pltpu
from jax.experimental.pallas import tpu_sc as plsc

B, L, D = 1, 8192, 1024
H = 16
DH = D // H
INNER = 1024
NUM_PROJS = 7
BLOCK = 256
SAMPLE = 256
NB = L // BLOCK
NCH = 32
CH = L // NCH
SCALE = DH ** -0.5
ROW = 2 * DH

_BF = jnp.bfloat16
_F32 = jnp.float32

_NC = 2
_LN = 16
GCH = 128


def _dot_bf16(a, b):
    return jnp.dot(a.astype(_BF), b.astype(_BF), preferred_element_type=_F32)


def _dotg_bf16(a, b):
    return lax.dot_general(a.astype(_BF), b.astype(_BF),
                           (((1,), (1,)), ((), ())),
                           preferred_element_type=_F32)


def _bins_from(s):
    lane = lax.broadcasted_iota(jnp.int32, (1, 128), 1)
    enc = jnp.where(lane < NUM_PROJS,
                    jnp.left_shift(1, jnp.where(lane < NUM_PROJS, lane, 0)), 0)
    bins = jnp.sum(jnp.where(s > 0, enc, 0), axis=1)
    return jnp.bitwise_xor(bins, jnp.right_shift(bins, 1))


def _qkv_kernel(x_ref, wq_ref, wkv_ref, bq_ref, bkv_ref, lsh_ref,
                oq_ref, okv_ref, obq_ref, obk_ref):
    h = pl.program_id(1)
    x = x_ref[...]
    oq = _dot_bf16(x, wq_ref[pl.ds(h, 1)][0]) + bq_ref[pl.ds(h, 1)][0]
    okv = _dot_bf16(x, wkv_ref[pl.ds(h, 1)][0]) + bkv_ref[pl.ds(h, 1)][0]
    oq_ref[0] = oq
    okv_ref[0] = okv
    obq_ref[0, 0, :] = _bins_from(_dot_bf16(oq, lsh_ref[...]))
    obk_ref[0, 0, :] = _bins_from(_dot_bf16(okv, lsh_ref[...]))


def _qkv_tables(x2d, wq, wkv, bq, bkv, lsh_pad):
    BM = 2048
    hb = pl.BlockSpec((H, D, ROW), lambda i, h: (0, 0, 0))
    bb = pl.BlockSpec((H, 1, ROW), lambda i, h: (0, 0, 0))
    ob = pl.BlockSpec((1, BM, ROW), lambda i, h: (h, i, 0))
    bins_b = pl.BlockSpec((1, 1, BM), lambda i, h: (h * (L // BM) + i, 0, 0))
    return pl.pallas_call(
        _qkv_kernel,
        grid=(L // BM, H),
        in_specs=[pl.BlockSpec((BM, D), lambda i, h: (i, 0)), hb, hb, bb, bb,
                  pl.BlockSpec((ROW, 128), lambda i, h: (0, 0))],
        out_specs=[ob, ob, bins_b, bins_b],
        out_shape=[jax.ShapeDtypeStruct((H, L, ROW), _F32),
                   jax.ShapeDtypeStruct((H, L, ROW), _F32),
                   jax.ShapeDtypeStruct((H * L // BM, 1, BM), jnp.int32),
                   jax.ShapeDtypeStruct((H * L // BM, 1, BM), jnp.int32)],
    )(x2d, wq, wkv, bq, bkv, lsh_pad)


_T = NCH * 128


def _sortpos_kernel(keys_ref, tri256_ref, e_ref, r_ref, v_ref, v2_ref,
                    triu32_ref, tri128_ref, vpat_ref, o_ref):
    keysf = keys_ref[0].astype(_F32)
    krep = jnp.dot(keysf.astype(_BF), e_ref[...],
                   preferred_element_type=_F32)
    oh = (krep == vpat_ref[...]).astype(_F32)
    ohb = oh.astype(_BF)
    rank_m = jnp.dot(tri256_ref[...], ohb, preferred_element_type=_F32)
    rank = jnp.dot((oh * rank_m).astype(_BF), r_ref[...],
                   preferred_element_type=_F32)
    hist_t = jnp.sum(oh, axis=0, keepdims=True)
    hist_vc = jnp.dot((v2_ref[...] * hist_t).astype(_BF), r_ref[...],
                      preferred_element_type=_F32)
    excl_vc = jnp.dot(hist_vc.astype(_BF), triu32_ref[...],
                      preferred_element_type=_F32)
    totalv = jnp.sum(hist_vc, axis=1, keepdims=True)
    key_off = jnp.dot(tri128_ref[...], totalv,
                      precision=lax.Precision.HIGHEST,
                      preferred_element_type=_F32)
    base_vc = key_off + excl_vc
    hi = jnp.floor(base_vc * (1.0 / 32.0))
    lo = base_vc - hi * 32.0
    bhi = jnp.dot(v_ref[...], hi.astype(_BF), preferred_element_type=_F32)
    blo = jnp.dot(v_ref[...], lo.astype(_BF), preferred_element_type=_F32)
    r_f = r_ref[...].astype(_F32)
    ph = jnp.dot(ohb, (bhi * r_f).astype(_BF), preferred_element_type=_F32)
    pll = jnp.dot(ohb, (blo * r_f).astype(_BF), preferred_element_type=_F32)
    pos = ph * 32.0 + pll + rank
    o_ref[0] = pos.astype(jnp.int32)


def _sort_positions(keysT):
    t_row = lax.broadcasted_iota(jnp.int32, (1, _T), 1)
    c_of_t = t_row >> 7
    v_of_t = t_row & 127
    e_m = ((t_row >> 7) == lax.broadcasted_iota(jnp.int32, (NCH, 1), 0)).astype(_BF)
    t_col = lax.broadcasted_iota(jnp.int32, (_T, 1), 0)
    r_m = ((t_col >> 7) == lax.broadcasted_iota(jnp.int32, (1, NCH), 1)).astype(_BF)
    v_m = ((t_col & 127) == lax.broadcasted_iota(jnp.int32, (1, 128), 1)).astype(_BF)
    v2_m = ((t_row & 127) == lax.broadcasted_iota(jnp.int32, (128, 1), 0)).astype(_F32)
    i256 = lax.broadcasted_iota(jnp.int32, (CH, CH), 0)
    j256 = lax.broadcasted_iota(jnp.int32, (CH, CH), 1)
    tri256 = (j256 < i256).astype(_BF)
    i32m = lax.broadcasted_iota(jnp.int32, (NCH, NCH), 0)
    j32m = lax.broadcasted_iota(jnp.int32, (NCH, NCH), 1)
    triu32 = (i32m < j32m).astype(_BF)
    i128 = lax.broadcasted_iota(jnp.int32, (128, 128), 0)
    j128 = lax.broadcasted_iota(jnp.int32, (128, 128), 1)
    tri128 = (j128 < i128).astype(_F32)
    vpat = v_of_t.astype(_F32)

    full = lambda shape: pl.BlockSpec(shape, lambda i: tuple(0 for _ in shape))
    return pl.pallas_call(
        _sortpos_kernel,
        grid=(2 * H,),
        in_specs=[pl.BlockSpec((1, CH, NCH), lambda i: (i, 0, 0)),
                  full((CH, CH)), full((NCH, _T)), full((_T, NCH)),
                  full((_T, 128)), full((128, _T)), full((NCH, NCH)),
                  full((128, 128)), full((1, _T))],
        out_specs=pl.BlockSpec((1, CH, NCH), lambda i: (i, 0, 0)),
        out_shape=jax.ShapeDtypeStruct((2 * H, CH, NCH), jnp.int32),
    )(keysT, tri256, e_m, r_m, v_m, v2_m, triu32, tri128, vpat)


def _sc_gather_sorted(table_q, table_kv, pos_flat, samp_flat):
    mesh = plsc.VectorSubcoreMesh(core_axis_name="c", subcore_axis_name="s")

    @functools.partial(
        pl.kernel, mesh=mesh,
        out_type=[jax.ShapeDtypeStruct((H, L, ROW), _F32),
                  jax.ShapeDtypeStruct((H, L, ROW), _F32),
                  jax.ShapeDtypeStruct((H, SAMPLE, ROW), _F32)],
        scratch_types=[pltpu.VMEM((L,), jnp.int32),
                       pltpu.VMEM((GCH,), jnp.int32),
                       pltpu.VMEM((GCH,), jnp.int32),
                       pltpu.VMEM((GCH, ROW), _F32),
                       pltpu.VMEM((GCH, ROW), _F32),
                       pltpu.VMEM((SAMPLE,), jnp.int32),
                       pltpu.SemaphoreType.DMA,
                       pltpu.SemaphoreType.DMA,
                       pltpu.SemaphoreType.DMA,
                       pltpu.SemaphoreType.DMA],
    )
    def k(tq_hbm, tkv_hbm, pos_hbm, samp_hbm, qs_hbm, kvs_hbm, sub_hbm,
          posbuf, idx_c0, idx_c1, rowbuf0, rowbuf1, sampbuf,
          semr0, semr1, semw0, semw1):
        wid = lax.axis_index("s") * _NC + lax.axis_index("c")
        h = lax.rem(wid, H)

        pltpu.sync_copy(pos_hbm.at[pl.ds(wid * L, L)], posbuf)

        def scatter_pair(table, out, j):
            c0, c1 = 2 * j, 2 * j + 1
            r0 = pltpu.async_copy(table.at[h].at[pl.ds(c0 * GCH, GCH)],
                                  rowbuf0, semr0)
            r1 = pltpu.async_copy(table.at[h].at[pl.ds(c1 * GCH, GCH)],
                                  rowbuf1, semr1)
            for r in range(GCH // _LN):
                idx_c0[pl.ds(r * _LN, _LN)] = \
                    posbuf[pl.ds(c0 * GCH + r * _LN, _LN)]
                idx_c1[pl.ds(r * _LN, _LN)] = \
                    posbuf[pl.ds(c1 * GCH + r * _LN, _LN)]
            r0.wait()
            w0 = pltpu.async_copy(rowbuf0, out.at[h].at[idx_c0], semw0)
            r1.wait()
            w1 = pltpu.async_copy(rowbuf1, out.at[h].at[idx_c1], semw1)
            w0.wait()
            w1.wait()

        @pl.when(wid < H)
        def _():
            lax.fori_loop(0, L // GCH // 2,
                          lambda j, _: (scatter_pair(tq_hbm, qs_hbm, j), 0)[1],
                          0)

        @pl.when(wid >= H)
        def _():
            lax.fori_loop(0, L // GCH // 2,
                          lambda j, _: (scatter_pair(tkv_hbm, kvs_hbm, j), 0)[1],
                          0)
            pltpu.sync_copy(samp_hbm.at[pl.ds(h * SAMPLE, SAMPLE)], sampbuf)
            for cc in range(SAMPLE // GCH):
                for r in range(GCH // _LN):
                    idx_c0[pl.ds(r * _LN, _LN)] = \
                        sampbuf[pl.ds(cc * GCH + r * _LN, _LN)]
                pltpu.async_copy(kvs_hbm.at[h].at[idx_c0], rowbuf0, semr0).wait()
                pltpu.sync_copy(rowbuf0, sub_hbm.at[h].at[pl.ds(cc * GCH, GCH)])

    return k(table_q, table_kv, pos_flat, samp_flat)


def _sc_unsort(attn, pos_flat):
    mesh = plsc.VectorSubcoreMesh(core_axis_name="c", subcore_axis_name="s")
    HALF = L // 2

    @functools.partial(
        pl.kernel, mesh=mesh,
        out_type=jax.ShapeDtypeStruct((H, L, ROW), _F32),
        scratch_types=[pltpu.VMEM((HALF,), jnp.int32),
                       pltpu.VMEM((GCH,), jnp.int32),
                       pltpu.VMEM((GCH,), jnp.int32),
                       pltpu.VMEM((GCH, ROW), _F32),
                       pltpu.VMEM((GCH, ROW), _F32),
                       pltpu.SemaphoreType.DMA,
                       pltpu.SemaphoreType.DMA,
                       pltpu.SemaphoreType.DMA,
                       pltpu.SemaphoreType.DMA],
    )
    def k(attn_hbm, pos_hbm, out_hbm, posbuf, idx_c0, idx_c1,
          rowbuf0, rowbuf1, semr0, semr1, semw0, semw1):
        wid = lax.axis_index("s") * _NC + lax.axis_index("c")
        h = lax.rem(wid, H)
        half = wid // H
        base = h * L + half * HALF
        pltpu.sync_copy(pos_hbm.at[pl.ds(base, HALF)], posbuf)

        def body(j, _):
            c0, c1 = 2 * j, 2 * j + 1
            for r in range(GCH // _LN):
                idx_c0[pl.ds(r * _LN, _LN)] = \
                    posbuf[pl.ds(c0 * GCH + r * _LN, _LN)]
                idx_c1[pl.ds(r * _LN, _LN)] = \
                    posbuf[pl.ds(c1 * GCH + r * _LN, _LN)]
            r0 = pltpu.async_copy(attn_hbm.at[h].at[idx_c0], rowbuf0, semr0)
            r1 = pltpu.async_copy(attn_hbm.at[h].at[idx_c1], rowbuf1, semr1)
            r0.wait()
            w0 = pltpu.async_copy(
                rowbuf0, out_hbm.at[h].at[pl.ds(half * HALF + c0 * GCH, GCH)],
                semw0)
            r1.wait()
            w1 = pltpu.async_copy(
                rowbuf1, out_hbm.at[h].at[pl.ds(half * HALF + c1 * GCH, GCH)],
                semw1)
            w0.wait()
            w1.wait()
            return 0
        lax.fori_loop(0, HALF // GCH // 2, body, 0)

    return k(attn, pos_flat)


_AB = 8


def _attn_kernel(q_ref, kv_ref, sub_ref, samp_ref, o_ref):
    sub = sub_ref[0]
    samp = samp_ref[0]
    sampb = samp // BLOCK
    for b in range(_AB):
        nb = pl.program_id(1) * _AB + b
        sl = pl.ds(b * BLOCK, BLOCK)
        qb = q_ref[0, sl, :DH]
        kv = kv_ref[0, sl, :]
        kb, vb = kv[:, :DH], kv[:, DH:]

        s1 = _dotg_bf16(qb, kb) * SCALE
        m1 = jnp.max(s1, axis=1, keepdims=True)
        e1 = jnp.exp(s1 - m1)
        l1 = jnp.log(jnp.sum(e1, axis=1, keepdims=True)) + m1
        a1 = jnp.exp(s1 - l1)
        o1 = _dot_bf16(a1, vb)

        bias = (sampb == nb).astype(_F32) * jnp.finfo(_F32).min
        s2 = _dotg_bf16(qb, sub[:, :DH]) * SCALE + bias
        m2 = jnp.max(s2, axis=1, keepdims=True)
        e2 = jnp.exp(s2 - m2)
        l2 = jnp.log(jnp.sum(e2, axis=1, keepdims=True)) + m2
        a2 = jnp.exp(s2 - l2)
        o2 = _dot_bf16(a2, sub[:, DH:])

        lse_res = l2 + math.log(L / SAMPLE)
        c = 1.0 / (1.0 + jnp.exp(lse_res - l1))
        o_ref[0, sl, :DH] = c * o1 + (1.0 - c) * o2
        o_ref[0, sl, DH:] = jnp.zeros((BLOCK, DH), _F32)


def _fused_attention(q_s, kv_s, kv_sub, samp):
    blk = pl.BlockSpec((1, _AB * BLOCK, ROW), lambda h, n: (h, n, 0))
    head = pl.BlockSpec((1, SAMPLE, ROW), lambda h, n: (h, 0, 0))
    return pl.pallas_call(
        _attn_kernel,
        grid=(H, NB // _AB),
        in_specs=[blk, blk, head,
                  pl.BlockSpec((1, 1, SAMPLE), lambda h, n: (h, 0, 0))],
        out_specs=blk,
        out_shape=jax.ShapeDtypeStruct((H, L, ROW), _F32),
    )(q_s, kv_s, kv_sub, samp)


def _proj_kernel(a_ref, w_ref, b_ref, o_ref):
    acc = _dot_bf16(a_ref[0], w_ref[0])
    for h in range(1, H):
        acc = acc + _dot_bf16(a_ref[h], w_ref[h])
    o_ref[...] = acc + b_ref[...]


def _proj_matmul(attn_u, w2, pb):
    BM = 1024
    return pl.pallas_call(
        _proj_kernel,
        grid=(L // BM,),
        in_specs=[pl.BlockSpec((H, BM, ROW), lambda i: (0, i, 0)),
                  pl.BlockSpec((H, ROW, D), lambda i: (0, 0, 0)),
                  pl.BlockSpec((1, D), lambda i: (0, 0))],
        out_specs=pl.BlockSpec((BM, D), lambda i: (i, 0)),
        out_shape=jax.ShapeDtypeStruct((L, D), _F32),
    )(attn_u, w2, pb)


def kernel(x, qkv_w, qkv_b, proj_w, proj_b, lsh_proj, sampled_set):
    x2d = x.reshape(L, D)

    zq = jnp.zeros((H, D, DH), _F32)
    wq = jnp.concatenate(
        [qkv_w[:INNER].reshape(H, DH, D).transpose(0, 2, 1), zq], axis=2)
    wkv = jnp.concatenate(
        [qkv_w[INNER:2 * INNER].reshape(H, DH, D).transpose(0, 2, 1),
         qkv_w[2 * INNER:].reshape(H, DH, D).transpose(0, 2, 1)], axis=2)
    bq = jnp.concatenate(
        [qkv_b[:INNER].reshape(H, 1, DH), jnp.zeros((H, 1, DH), _F32)], axis=2)
    bkv = jnp.concatenate(
        [qkv_b[INNER:2 * INNER].reshape(H, 1, DH),
         qkv_b[2 * INNER:].reshape(H, 1, DH)], axis=2)
    lsh_pad = jnp.concatenate(
        [jnp.concatenate([lsh_proj, jnp.zeros((DH, NUM_PROJS), _F32)], axis=0),
         jnp.zeros((ROW, 128 - NUM_PROJS), _F32)], axis=1)
    w2 = jnp.concatenate(
        [proj_w.T.reshape(H, DH, D), jnp.zeros((H, DH, D), _F32)],
        axis=1).astype(_BF)

    table_q, table_kv, bins_q, bins_k = _qkv_tables(x2d, wq, wkv, bq, bkv,
                                                    lsh_pad)
    bins = jnp.concatenate([bins_q, bins_k], axis=0)
    keysT = bins.reshape(2 * H, NCH, CH).transpose(0, 2, 1)
    posT = _sort_positions(keysT)
    pos_flat = posT.transpose(0, 2, 1).reshape(2 * H * L)

    samp = sampled_set[0].astype(jnp.int32)
    samp_flat = samp.reshape(H * SAMPLE)

    q_s, kv_s, kv_sub = _sc_gather_sorted(table_q, table_kv,
                                          pos_flat, samp_flat)

    attn = _fused_attention(q_s, kv_s, kv_sub, samp.reshape(H, 1, SAMPLE))

    attn_u = _sc_unsort(attn, pos_flat)

    out = _proj_matmul(attn_u, w2, proj_b.reshape(1, D))
    return out.reshape(B, L, D)
